# baseline (speedup 1.0000x reference)
import functools

import jax
import jax.numpy as jnp
from jax import lax
from jax.experimental import pallas as pl
from jax.experimental.pallas import tpu as pltpu
from jax.experimental.pallas import tpu_sc as plsc

BATCH = 16384
DIM = 64
PAIR = 2 * DIM

_info = plsc.get_sparse_core_info()
NC, NS, L = _info.num_cores, _info.num_subcores, _info.num_lanes
NW = NC * NS
BPW = BATCH // NW
CHUNK = 128
NCHUNK = BPW // CHUNK
CGROUPS = CHUNK // L


def _sc_body(pid_hbm, sid_hbm, ptab_hbm, stab_hbm, w_hbm, b_hbm, out_hbm,
             idp_v, ids_v, idx2p_v, idx2s_v, hofp_v, hofs_v,
             rows_p, rows_s, w_v, b_v, out_v, sem_p, sem_s):
    wid = lax.axis_index("s") * NC + lax.axis_index("c")
    base = wid * BPW

    pltpu.sync_copy(pid_hbm.at[pl.ds(base, BPW)], idp_v)
    pltpu.sync_copy(sid_hbm.at[pl.ds(base, BPW)], ids_v)
    pltpu.sync_copy(w_hbm, w_v)
    pltpu.sync_copy(b_hbm, b_v)

    bias = b_v[...]
    wp = [w_v[pl.ds(m * L, L)] for m in range(DIM // L)]
    ws = [w_v[pl.ds(DIM + m * L, L)] for m in range(DIM // L)]
    lane = lax.iota(jnp.int32, L)
    cols = [lane + m * L for m in range(DIM // L)]

    def chunk_step(c, carry):
        c0 = c * CHUNK
        def idx_prep(u, carry2):
            s = c0 + u * L
            vpi = idp_v[pl.ds(s, L)]
            vsi = ids_v[pl.ds(s, L)]
            d = u * L
            idx2p_v[pl.ds(d, L)] = lax.shift_right_logical(vpi, 1)
            idx2s_v[pl.ds(d, L)] = lax.shift_right_logical(vsi, 1)
            hofp_v[pl.ds(d, L)] = lax.shift_left(vpi & 1, 6)
            hofs_v[pl.ds(d, L)] = lax.shift_left(vsi & 1, 6)
            return carry2
        lax.fori_loop(0, CHUNK // L, idx_prep, 0)

        cp = pltpu.async_copy(ptab_hbm.at[idx2p_v], rows_p, sem_p)
        cs = pltpu.async_copy(stab_hbm.at[idx2s_v], rows_s, sem_s)
        cp.wait()
        cs.wait()

        def group(gl, carry2):
            r0 = gl * L
            hp = hofp_v[pl.ds(r0, L)]
            hs = hofs_v[pl.ds(r0, L)]
            logits = bias
            for j in range(L):
                r = r0 + j
                rvec = jnp.full((L,), r, jnp.int32)
                cp0 = cols[0] + hp[j]
                cs0 = cols[0] + hs[j]
                acc = (plsc.load_gather(rows_p, [rvec, cp0]) * wp[0]
                       + plsc.load_gather(rows_s, [rvec, cs0]) * ws[0])
                for m in range(1, DIM // L):
                    cpm = cols[m] + hp[j]
                    csm = cols[m] + hs[j]
                    acc = acc + plsc.load_gather(rows_p, [rvec, cpm]) * wp[m]
                    acc = acc + plsc.load_gather(rows_s, [rvec, csm]) * ws[m]
                tot = jnp.sum(acc)
                logits = jnp.where(lane == j, logits + tot, logits)
            pred = 1.0 / (1.0 + jnp.exp(-logits))
            out_v[pl.ds(c0 + r0, L)] = pred
            return carry2

        lax.fori_loop(0, CGROUPS, group, 0)
        return carry

    lax.fori_loop(0, NCHUNK, chunk_step, 0)
    pltpu.sync_copy(out_v, out_hbm.at[pl.ds(base, BPW)])


@jax.jit
def _run(playlist_ids, song_ids, ptab2, stab2, w_flat, b_vec):
    mesh = plsc.VectorSubcoreMesh(core_axis_name="c", subcore_axis_name="s")
    call = functools.partial(
        pl.kernel,
        mesh=mesh,
        compiler_params=pltpu.CompilerParams(needs_layout_passes=False),
        out_type=jax.ShapeDtypeStruct((BATCH,), jnp.float32),
        scratch_types=[
            pltpu.VMEM((BPW,), jnp.int32),
            pltpu.VMEM((BPW,), jnp.int32),
            pltpu.VMEM((CHUNK,), jnp.int32),
            pltpu.VMEM((CHUNK,), jnp.int32),
            pltpu.VMEM((CHUNK,), jnp.int32),
            pltpu.VMEM((CHUNK,), jnp.int32),
            pltpu.VMEM((CHUNK, PAIR), jnp.float32),
            pltpu.VMEM((CHUNK, PAIR), jnp.float32),
            pltpu.VMEM((PAIR,), jnp.float32),
            pltpu.VMEM((L,), jnp.float32),
            pltpu.VMEM((BPW,), jnp.float32),
            pltpu.SemaphoreType.DMA,
            pltpu.SemaphoreType.DMA,
        ],
    )(_sc_body)
    return call(playlist_ids, song_ids, ptab2, stab2, w_flat, b_vec)


def kernel(playlist_ids, song_ids, playlist_table, song_table, fc_w, fc_b):
    ptab2 = playlist_table.reshape(-1, PAIR)
    stab2 = song_table.reshape(-1, PAIR)
    w_flat = fc_w.reshape(2 * DIM)
    b_vec = jnp.broadcast_to(fc_b.astype(jnp.float32), (L,))
    out = _run(playlist_ids, song_ids, ptab2, stab2, w_flat, b_vec)
    return out.reshape(BATCH, 1)


# speedup vs baseline: 1.6377x; 1.6377x over previous
import functools

import jax
import jax.numpy as jnp
from jax import lax
from jax.experimental import pallas as pl
from jax.experimental.pallas import tpu as pltpu
from jax.experimental.pallas import tpu_sc as plsc

BATCH = 16384
DIM = 64

_info = plsc.get_sparse_core_info()
NC, NS, L = _info.num_cores, _info.num_subcores, _info.num_lanes
NW = NC * NS
BPW = BATCH // NW
CHUNK = 64
NCHUNK = BPW // CHUNK
CGROUPS = CHUNK // L


def _sc_body(pid_hbm, sid_hbm, ptab_hbm, stab_hbm, w_hbm, b_hbm, out_hbm,
             idp_v, ids_v, rows_p, rows_s, w_v, b_v, out_v, sem_p, sem_s):
    wid = lax.axis_index("s") * NC + lax.axis_index("c")
    base = wid * BPW

    pltpu.sync_copy(pid_hbm.at[pl.ds(base, BPW)], idp_v)
    pltpu.sync_copy(sid_hbm.at[pl.ds(base, BPW)], ids_v)
    pltpu.sync_copy(w_hbm, w_v)
    pltpu.sync_copy(b_hbm, b_v)

    bias = b_v[...]
    wp = [w_v[pl.ds(m * L, L)] for m in range(DIM // L)]
    ws = [w_v[pl.ds(DIM + m * L, L)] for m in range(DIM // L)]
    lane = lax.iota(jnp.int32, L)

    def chunk_step(c, carry):
        c0 = c * CHUNK
        copies = []
        for u in range(CGROUPS):
            vpi = idp_v[pl.ds(c0 + u * L, L)]
            vsi = ids_v[pl.ds(c0 + u * L, L)]
            for j in range(L):
                r = u * L + j
                copies.append(pltpu.async_copy(
                    ptab_hbm.at[pl.ds(vpi[j], 1), :],
                    rows_p.at[pl.ds(r, 1), :], sem_p))
                copies.append(pltpu.async_copy(
                    stab_hbm.at[pl.ds(vsi[j], 1), :],
                    rows_s.at[pl.ds(r, 1), :], sem_s))
        for cp in copies:
            cp.wait()

        def group(gl, carry2):
            r0 = gl * L
            logits = bias
            for j in range(L):
                r = r0 + j
                acc = rows_p[r, pl.ds(0, L)] * wp[0]
                for m in range(1, DIM // L):
                    acc = acc + rows_p[r, pl.ds(m * L, L)] * wp[m]
                for m in range(DIM // L):
                    acc = acc + rows_s[r, pl.ds(m * L, L)] * ws[m]
                tot = jnp.sum(acc)
                logits = jnp.where(lane == j, logits + tot, logits)
            pred = 1.0 / (1.0 + jnp.exp(-logits))
            out_v[pl.ds(c0 + r0, L)] = pred
            return carry2

        lax.fori_loop(0, CGROUPS, group, 0)
        return carry

    lax.fori_loop(0, NCHUNK, chunk_step, 0)
    pltpu.sync_copy(out_v, out_hbm.at[pl.ds(base, BPW)])


@jax.jit
def _run(playlist_ids, song_ids, playlist_table, song_table, w_flat, b_vec):
    mesh = plsc.VectorSubcoreMesh(core_axis_name="c", subcore_axis_name="s")
    call = functools.partial(
        pl.kernel,
        mesh=mesh,
        compiler_params=pltpu.CompilerParams(needs_layout_passes=False),
        out_type=jax.ShapeDtypeStruct((BATCH,), jnp.float32),
        scratch_types=[
            pltpu.VMEM((BPW,), jnp.int32),
            pltpu.VMEM((BPW,), jnp.int32),
            pltpu.VMEM((CHUNK, DIM), jnp.float32),
            pltpu.VMEM((CHUNK, DIM), jnp.float32),
            pltpu.VMEM((2 * DIM,), jnp.float32),
            pltpu.VMEM((L,), jnp.float32),
            pltpu.VMEM((BPW,), jnp.float32),
            pltpu.SemaphoreType.DMA,
            pltpu.SemaphoreType.DMA,
        ],
    )(_sc_body)
    return call(playlist_ids, song_ids, playlist_table, song_table,
                w_flat, b_vec)


def kernel(playlist_ids, song_ids, playlist_table, song_table, fc_w, fc_b):
    w_flat = fc_w.reshape(2 * DIM)
    b_vec = jnp.broadcast_to(fc_b.astype(jnp.float32), (L,))
    out = _run(playlist_ids, song_ids, playlist_table, song_table,
               w_flat, b_vec)
    return out.reshape(BATCH, 1)


# speedup vs baseline: 1.8620x; 1.1369x over previous
import functools

import jax
import jax.numpy as jnp
from jax import lax
from jax.experimental import pallas as pl
from jax.experimental.pallas import tpu as pltpu
from jax.experimental.pallas import tpu_sc as plsc

BATCH = 16384
DIM = 64

_info = plsc.get_sparse_core_info()
NC, NS, L = _info.num_cores, _info.num_subcores, _info.num_lanes
NW = NC * NS
BPW = BATCH // NW

BN = 2048


def _matvec_body(w_ref, x_ref, o_ref):
    x = x_ref[...]
    w = w_ref[...]
    o_ref[...] = jnp.sum(x * w, axis=0)


def _scores(tabT, w_col):
    n = tabT.shape[1]
    grid = (n + BN - 1) // BN
    return pl.pallas_call(
        _matvec_body,
        grid=(grid,),
        in_specs=[
            pl.BlockSpec((DIM, 1), lambda i: (0, 0)),
            pl.BlockSpec((DIM, BN), lambda i: (0, i)),
        ],
        out_specs=pl.BlockSpec((BN,), lambda i: (i,)),
        out_shape=jax.ShapeDtypeStruct((n,), jnp.float32),
    )(w_col, tabT)


def _sc_body(pid_hbm, sid_hbm, sp_hbm, ss_hbm, b_hbm, out_hbm,
             idp_v, ids_v, gp_v, gs_v, b_v, out_v, sem_p, sem_s):
    wid = lax.axis_index("s") * NC + lax.axis_index("c")
    base = wid * BPW

    pltpu.sync_copy(pid_hbm.at[pl.ds(base, BPW)], idp_v)
    pltpu.sync_copy(sid_hbm.at[pl.ds(base, BPW)], ids_v)
    pltpu.sync_copy(b_hbm, b_v)
    cp = pltpu.async_copy(sp_hbm.at[idp_v], gp_v, sem_p)
    cs = pltpu.async_copy(ss_hbm.at[ids_v], gs_v, sem_s)
    cp.wait()
    cs.wait()

    bias = b_v[...]

    def group(g, carry):
        s0 = g * L
        logits = gp_v[pl.ds(s0, L)] + gs_v[pl.ds(s0, L)] + bias
        out_v[pl.ds(s0, L)] = 1.0 / (1.0 + jnp.exp(-logits))
        return carry

    lax.fori_loop(0, BPW // L, group, 0)
    pltpu.sync_copy(out_v, out_hbm.at[pl.ds(base, BPW)])


def _gather_sigmoid(playlist_ids, song_ids, scores_p, scores_s, b_vec):
    mesh = plsc.VectorSubcoreMesh(core_axis_name="c", subcore_axis_name="s")
    call = functools.partial(
        pl.kernel,
        mesh=mesh,
        compiler_params=pltpu.CompilerParams(needs_layout_passes=False),
        out_type=jax.ShapeDtypeStruct((BATCH,), jnp.float32),
        scratch_types=[
            pltpu.VMEM((BPW,), jnp.int32),
            pltpu.VMEM((BPW,), jnp.int32),
            pltpu.VMEM((BPW,), jnp.float32),
            pltpu.VMEM((BPW,), jnp.float32),
            pltpu.VMEM((L,), jnp.float32),
            pltpu.VMEM((BPW,), jnp.float32),
            pltpu.SemaphoreType.DMA,
            pltpu.SemaphoreType.DMA,
        ],
    )(_sc_body)
    return call(playlist_ids, song_ids, scores_p, scores_s, b_vec)


@jax.jit
def _run(playlist_ids, song_ids, ptabT, stabT, fc_w, fc_b):
    w1 = fc_w[:DIM]
    w2 = fc_w[DIM:]
    scores_p = _scores(ptabT, w1)
    scores_s = _scores(stabT, w2)
    b_vec = jnp.broadcast_to(fc_b.astype(jnp.float32), (L,))
    return _gather_sigmoid(playlist_ids, song_ids, scores_p, scores_s, b_vec)


def kernel(playlist_ids, song_ids, playlist_table, song_table, fc_w, fc_b):
    out = _run(playlist_ids, song_ids, playlist_table.T, song_table.T,
               fc_w, fc_b)
    return out.reshape(BATCH, 1)


# speedup vs baseline: 2.8745x; 1.5438x over previous
import functools

import jax
import jax.numpy as jnp
from jax import lax
from jax.experimental import pallas as pl
from jax.experimental.pallas import tpu as pltpu
from jax.experimental.pallas import tpu_sc as plsc

BATCH = 16384
DIM = 64

_info = plsc.get_sparse_core_info()
NC, NS, L = _info.num_cores, _info.num_subcores, _info.num_lanes
NW = NC * NS
BPW = BATCH // NW

BN = 4096


def _matvec_body(w_ref, x_ref, o_ref):
    x = x_ref[...]
    wt = w_ref[...].T
    res = jax.lax.dot_general(wt, x, (((1,), (0,)), ((), ())),
                              preferred_element_type=jnp.float32)
    o_ref[...] = res[0]


def _scores(tabT, w_col):
    n = tabT.shape[1]
    grid = (n + BN - 1) // BN
    return pl.pallas_call(
        _matvec_body,
        grid=(grid,),
        in_specs=[
            pl.BlockSpec((DIM, 1), lambda i: (0, 0)),
            pl.BlockSpec((DIM, BN), lambda i: (0, i)),
        ],
        out_specs=pl.BlockSpec((BN,), lambda i: (i,)),
        out_shape=jax.ShapeDtypeStruct((n,), jnp.float32),
    )(w_col, tabT)


def _sc_body(pid_hbm, sid_hbm, sp_hbm, ss_hbm, b_hbm, out_hbm,
             idp_v, ids_v, gp_v, gs_v, b_v, out_v, sem_p, sem_s):
    wid = lax.axis_index("s") * NC + lax.axis_index("c")
    base = wid * BPW

    pltpu.sync_copy(pid_hbm.at[pl.ds(base, BPW)], idp_v)
    pltpu.sync_copy(sid_hbm.at[pl.ds(base, BPW)], ids_v)
    pltpu.sync_copy(b_hbm, b_v)
    cp = pltpu.async_copy(sp_hbm.at[idp_v], gp_v, sem_p)
    cs = pltpu.async_copy(ss_hbm.at[ids_v], gs_v, sem_s)
    cp.wait()
    cs.wait()

    bias = b_v[...]

    def group(g, carry):
        s0 = g * L
        logits = gp_v[pl.ds(s0, L)] + gs_v[pl.ds(s0, L)] + bias
        out_v[pl.ds(s0, L)] = 1.0 / (1.0 + jnp.exp(-logits))
        return carry

    lax.fori_loop(0, BPW // L, group, 0)
    pltpu.sync_copy(out_v, out_hbm.at[pl.ds(base, BPW)])


def _gather_sigmoid(playlist_ids, song_ids, scores_p, scores_s, b_vec):
    mesh = plsc.VectorSubcoreMesh(core_axis_name="c", subcore_axis_name="s")
    call = functools.partial(
        pl.kernel,
        mesh=mesh,
        compiler_params=pltpu.CompilerParams(needs_layout_passes=False),
        out_type=jax.ShapeDtypeStruct((BATCH,), jnp.float32),
        scratch_types=[
            pltpu.VMEM((BPW,), jnp.int32),
            pltpu.VMEM((BPW,), jnp.int32),
            pltpu.VMEM((BPW,), jnp.float32),
            pltpu.VMEM((BPW,), jnp.float32),
            pltpu.VMEM((L,), jnp.float32),
            pltpu.VMEM((BPW,), jnp.float32),
            pltpu.SemaphoreType.DMA,
            pltpu.SemaphoreType.DMA,
        ],
    )(_sc_body)
    return call(playlist_ids, song_ids, scores_p, scores_s, b_vec)


@jax.jit
def _run(playlist_ids, song_ids, ptabT, stabT, fc_w, fc_b):
    w1 = fc_w[:DIM]
    w2 = fc_w[DIM:]
    scores_p = _scores(ptabT, w1)
    scores_s = _scores(stabT, w2)
    b_vec = jnp.broadcast_to(fc_b.astype(jnp.float32), (L,))
    return _gather_sigmoid(playlist_ids, song_ids, scores_p, scores_s, b_vec)


def kernel(playlist_ids, song_ids, playlist_table, song_table, fc_w, fc_b):
    out = _run(playlist_ids, song_ids, playlist_table.T, song_table.T,
               fc_w, fc_b)
    return out.reshape(BATCH, 1)


# speedup vs baseline: 5.4250x; 1.8873x over previous
import functools

import jax
import jax.numpy as jnp
from jax import lax
from jax.experimental import pallas as pl
from jax.experimental.pallas import tpu as pltpu
from jax.experimental.pallas import tpu_sc as plsc

BATCH = 16384
DIM = 64

_info = plsc.get_sparse_core_info()
NC, NS, L = _info.num_cores, _info.num_subcores, _info.num_lanes
NW = NC * NS
BPW = BATCH // NW

BN = 16384


def _matvec_body(w_ref, x_ref, o_ref):
    x = x_ref[...]
    wt = w_ref[...].T
    res = jax.lax.dot_general(wt, x, (((1,), (0,)), ((), ())),
                              preferred_element_type=jnp.float32)
    o_ref[...] = res[0]


def _scores(tabT, w_col):
    n = tabT.shape[1]
    grid = (n + BN - 1) // BN
    return pl.pallas_call(
        _matvec_body,
        grid=(grid,),
        in_specs=[
            pl.BlockSpec((DIM, 1), lambda i: (0, 0)),
            pl.BlockSpec((DIM, BN), lambda i: (0, i)),
        ],
        out_specs=pl.BlockSpec((BN,), lambda i: (i,)),
        out_shape=jax.ShapeDtypeStruct((n,), jnp.float32),
    )(w_col, tabT)


def _sc_body(pid_hbm, sid_hbm, sp_hbm, ss_hbm, b_hbm, out_hbm,
             idp_v, ids_v, gp_v, gs_v, b_v, out_v, sem_p, sem_s):
    wid = lax.axis_index("s") * NC + lax.axis_index("c")
    base = wid * BPW

    pltpu.sync_copy(pid_hbm.at[pl.ds(base, BPW)], idp_v)
    pltpu.sync_copy(sid_hbm.at[pl.ds(base, BPW)], ids_v)
    pltpu.sync_copy(b_hbm, b_v)
    cp = pltpu.async_copy(sp_hbm.at[idp_v], gp_v, sem_p)
    cs = pltpu.async_copy(ss_hbm.at[ids_v], gs_v, sem_s)
    cp.wait()
    cs.wait()

    bias = b_v[...]

    def group(g, carry):
        s0 = g * L
        logits = gp_v[pl.ds(s0, L)] + gs_v[pl.ds(s0, L)] + bias
        out_v[pl.ds(s0, L)] = 1.0 / (1.0 + jnp.exp(-logits))
        return carry

    lax.fori_loop(0, BPW // L, group, 0)
    pltpu.sync_copy(out_v, out_hbm.at[pl.ds(base, BPW)])


def _gather_sigmoid(playlist_ids, song_ids, scores_p, scores_s, b_vec):
    mesh = plsc.VectorSubcoreMesh(core_axis_name="c", subcore_axis_name="s")
    call = functools.partial(
        pl.kernel,
        mesh=mesh,
        compiler_params=pltpu.CompilerParams(needs_layout_passes=False),
        out_type=jax.ShapeDtypeStruct((BATCH,), jnp.float32),
        scratch_types=[
            pltpu.VMEM((BPW,), jnp.int32),
            pltpu.VMEM((BPW,), jnp.int32),
            pltpu.VMEM((BPW,), jnp.float32),
            pltpu.VMEM((BPW,), jnp.float32),
            pltpu.VMEM((L,), jnp.float32),
            pltpu.VMEM((BPW,), jnp.float32),
            pltpu.SemaphoreType.DMA,
            pltpu.SemaphoreType.DMA,
        ],
    )(_sc_body)
    return call(playlist_ids, song_ids, scores_p, scores_s, b_vec)


@jax.jit
def _run(playlist_ids, song_ids, ptabT, stabT, fc_w, fc_b):
    w1 = fc_w[:DIM]
    w2 = fc_w[DIM:]
    scores_p = _scores(ptabT, w1)
    scores_s = _scores(stabT, w2)
    b_vec = jnp.broadcast_to(fc_b.astype(jnp.float32), (L,))
    return _gather_sigmoid(playlist_ids, song_ids, scores_p, scores_s, b_vec)


def kernel(playlist_ids, song_ids, playlist_table, song_table, fc_w, fc_b):
    out = _run(playlist_ids, song_ids, playlist_table.T, song_table.T,
               fc_w, fc_b)
    return out.reshape(BATCH, 1)


# speedup vs baseline: 6.0490x; 1.1150x over previous
import functools

import jax
import jax.numpy as jnp
from jax import lax
from jax.experimental import pallas as pl
from jax.experimental.pallas import tpu as pltpu
from jax.experimental.pallas import tpu_sc as plsc

BATCH = 16384
DIM = 64

_info = plsc.get_sparse_core_info()
NC, NS, L = _info.num_cores, _info.num_subcores, _info.num_lanes
NW = NC * NS
BPW = BATCH // NW

BN = 32768


def _matvec_body(w_ref, x_ref, o_ref):
    x = x_ref[...]
    wt = w_ref[...].T
    res = jax.lax.dot_general(wt, x, (((1,), (0,)), ((), ())),
                              preferred_element_type=jnp.float32)
    o_ref[...] = res[0]


def _scores(tabT, w_col):
    n = tabT.shape[1]
    grid = (n + BN - 1) // BN
    return pl.pallas_call(
        _matvec_body,
        grid=(grid,),
        in_specs=[
            pl.BlockSpec((DIM, 1), lambda i: (0, 0)),
            pl.BlockSpec((DIM, BN), lambda i: (0, i)),
        ],
        out_specs=pl.BlockSpec((BN,), lambda i: (i,)),
        out_shape=jax.ShapeDtypeStruct((n,), jnp.float32),
    )(w_col, tabT)


def _sc_body(pid_hbm, sid_hbm, sp_hbm, ss_hbm, b_hbm, out_hbm,
             idp_v, ids_v, gp_v, gs_v, b_v, out_v, sem_p, sem_s):
    wid = lax.axis_index("s") * NC + lax.axis_index("c")
    base = wid * BPW

    pltpu.sync_copy(pid_hbm.at[pl.ds(base, BPW)], idp_v)
    pltpu.sync_copy(sid_hbm.at[pl.ds(base, BPW)], ids_v)
    pltpu.sync_copy(b_hbm, b_v)
    cp = pltpu.async_copy(sp_hbm.at[idp_v], gp_v, sem_p)
    cs = pltpu.async_copy(ss_hbm.at[ids_v], gs_v, sem_s)
    cp.wait()
    cs.wait()

    bias = b_v[...]

    def group(g, carry):
        s0 = g * L
        logits = gp_v[pl.ds(s0, L)] + gs_v[pl.ds(s0, L)] + bias
        out_v[pl.ds(s0, L)] = 1.0 / (1.0 + jnp.exp(-logits))
        return carry

    lax.fori_loop(0, BPW // L, group, 0)
    pltpu.sync_copy(out_v, out_hbm.at[pl.ds(base, BPW)])


def _gather_sigmoid(playlist_ids, song_ids, scores_p, scores_s, b_vec):
    mesh = plsc.VectorSubcoreMesh(core_axis_name="c", subcore_axis_name="s")
    call = functools.partial(
        pl.kernel,
        mesh=mesh,
        compiler_params=pltpu.CompilerParams(needs_layout_passes=False),
        out_type=jax.ShapeDtypeStruct((BATCH,), jnp.float32),
        scratch_types=[
            pltpu.VMEM((BPW,), jnp.int32),
            pltpu.VMEM((BPW,), jnp.int32),
            pltpu.VMEM((BPW,), jnp.float32),
            pltpu.VMEM((BPW,), jnp.float32),
            pltpu.VMEM((L,), jnp.float32),
            pltpu.VMEM((BPW,), jnp.float32),
            pltpu.SemaphoreType.DMA,
            pltpu.SemaphoreType.DMA,
        ],
    )(_sc_body)
    return call(playlist_ids, song_ids, scores_p, scores_s, b_vec)


@jax.jit
def _run(playlist_ids, song_ids, ptabT, stabT, fc_w, fc_b):
    w1 = fc_w[:DIM]
    w2 = fc_w[DIM:]
    scores_p = _scores(ptabT, w1)
    scores_s = _scores(stabT, w2)
    b_vec = jnp.broadcast_to(fc_b.astype(jnp.float32), (L,))
    return _gather_sigmoid(playlist_ids, song_ids, scores_p, scores_s, b_vec)


def kernel(playlist_ids, song_ids, playlist_table, song_table, fc_w, fc_b):
    out = _run(playlist_ids, song_ids, playlist_table.T, song_table.T,
               fc_w, fc_b)
    return out.reshape(BATCH, 1)
